# Initial kernel scaffold; baseline (speedup 1.0000x reference)
import jax
import jax.numpy as jnp
from jax.experimental import pallas as pl


def kernel(X, V, rho, W1, b1, W2, b2, W3, b3, W4, b4, W1r, b1r, W2r, b2r, W3r, b3r, W4r, b4r, alpha1, alpha2, beta1, beta2, neighbor, batch):
    raise NotImplementedError("write your pallas kernel here")


# speedup vs baseline: 1.0253x; 1.0253x over previous
import functools

import jax
import jax.numpy as jnp
import numpy as np
from jax import lax
from jax.experimental import pallas as pl
from jax.experimental.pallas import tpu as pltpu
from jax.experimental.pallas import tpu_sc as plsc

N = 50000
L = 32
NNB = L - 1
E = N * NNB
PI = 3.14159265358
H = float(((2.0 * np.pi) ** 3 / N * L / np.pi / (4.0 / 3.0)) ** (1.0 / 3.0))

NW = 32
EPW = 48440
CH = 9688
NCH = 5

BN = 2000
NB = N // BN


def _sc_gather(table, idx):
    mesh = plsc.VectorSubcoreMesh(core_axis_name="c", subcore_axis_name="s")

    @functools.partial(
        pl.kernel,
        out_type=jax.ShapeDtypeStruct((E, 8), jnp.float32),
        mesh=mesh,
        scratch_types=[
            pltpu.VMEM((CH,), jnp.int32),
            pltpu.VMEM((CH, 8), jnp.float32),
            pltpu.SemaphoreType.DMA,
        ],
        compiler_params=pltpu.CompilerParams(use_tc_tiling_on_sc=False),
    )
    def k(table_hbm, idx_hbm, out_hbm, idx_v, rows_v, sem):
        wid = lax.axis_index("s") * 2 + lax.axis_index("c")
        s0 = wid * EPW

        def body(c, carry):
            base = jnp.minimum(s0 + c * CH, E - CH)
            pltpu.sync_copy(idx_hbm.at[pl.ds(base, CH)], idx_v)
            pltpu.async_copy(table_hbm.at[idx_v], rows_v, sem).wait()
            pltpu.sync_copy(rows_v, out_hbm.at[pl.ds(base, CH)])
            return carry

        lax.fori_loop(0, NCH, body, 0)

    return k(table, idx)


def _tc_body(g_ref, t_ref, w1_ref, b1_ref, w2_ref, b2_ref, w3_ref, b3_ref,
             w4_ref, b4_ref, s_ref, o_ref):
    i = pl.program_id(1)
    g = g_ref[0]
    t = t_ref[...]

    d = t[:, 0:3] - g[:, 0:3]
    temp1 = jnp.abs(d)
    sgn = -jnp.sign(d) * jnp.sign(d + PI) * jnp.sign(d - PI)
    out = sgn * jnp.minimum(temp1, 2.0 * PI - temp1) / H
    outv = t[:, 3:6] - g[:, 3:6]
    out2 = jnp.sum(out * out, axis=1, keepdims=True)
    outv2 = jnp.sum(outv * outv, axis=1, keepdims=True)
    out2v = jnp.sum(out * outv, axis=1, keepdims=True)
    drho1 = t[:, 6:7]
    drho2 = g[:, 6:7]
    sq_out2 = jnp.sqrt(out2)
    sq_outv2 = jnp.sqrt(outv2)

    feat = jnp.concatenate(
        [drho1, drho2, sq_out2, sq_outv2, out2v,
         jnp.zeros((BN, 3), jnp.float32)], axis=1)
    r = drho1 - drho2
    dis0 = r / jnp.abs(r)
    disA = out / sq_out2
    disB = outv / sq_outv2

    h1 = jnp.tanh(jnp.dot(feat, w1_ref[...],
                          preferred_element_type=jnp.float32) + b1_ref[...])
    h2 = jnp.tanh(jnp.dot(h1, w2_ref[...],
                          preferred_element_type=jnp.float32) + b2_ref[...])
    h3 = jnp.tanh(jnp.dot(h2, w3_ref[...],
                          preferred_element_type=jnp.float32) + b3_ref[...])
    h4 = jnp.dot(h3, w4_ref[...],
                 preferred_element_type=jnp.float32) + b4_ref[...]

    drho0 = h4[:, 2:3] * dis0
    drhov = h4[:, 0:1] * disA + h4[:, 1:2] * disB

    a1 = jnp.abs(s_ref[0])
    a2 = jnp.abs(s_ref[1])
    bt1 = jnp.abs(s_ref[2])
    bt2 = jnp.abs(s_ref[3])
    denom = out2 + 0.1 * H * H
    out_rho = r * (H * H) / denom
    out_rho = -(bt1 + bt2 * jnp.abs(out_rho)) * out_rho
    o = -1.0 * H * jnp.tanh(-1.0 * out2v) / denom
    o = -a1 * o + a2 * o * o
    drho0 = drho0 + out_rho
    drhov = drhov + o * disA

    delta = jnp.concatenate([drho0, drhov], axis=1)

    @pl.when(i == 0)
    def _():
        o_ref[...] = delta

    @pl.when(i != 0)
    def _():
        o_ref[...] = o_ref[...] + delta


def kernel(X, V, rho, W1, b1, W2, b2, W3, b3, W4, b4, W1r, b1r, W2r, b2r,
           W3r, b3r, W4r, b4r, alpha1, alpha2, beta1, beta2, neighbor, batch):
    del batch

    table = jnp.concatenate([X, V, rho, jnp.zeros((N, 1), jnp.float32)],
                            axis=1)
    idx = jnp.transpose(neighbor[:, 1:]).reshape(-1)

    g_flat = _sc_gather(table, idx)
    g = g_flat.reshape(NNB, N, 8)

    w1c = jnp.zeros((8, 40), jnp.float32)
    w1c = w1c.at[0:5, 0:20].set(W1).at[0:5, 20:40].set(W1r)
    b1c = jnp.concatenate([b1, b1r]).reshape(1, 40)
    w2c = jnp.zeros((40, 200), jnp.float32)
    w2c = w2c.at[0:20, 0:100].set(W2).at[20:40, 100:200].set(W2r)
    b2c = jnp.concatenate([b2, b2r]).reshape(1, 200)
    w3c = jnp.zeros((200, 40), jnp.float32)
    w3c = w3c.at[0:100, 0:20].set(W3).at[100:200, 20:40].set(W3r)
    b3c = jnp.concatenate([b3, b3r]).reshape(1, 40)
    w4c = jnp.zeros((40, 3), jnp.float32)
    w4c = w4c.at[0:20, 0:2].set(W4).at[20:40, 2:3].set(W4r)
    b4c = jnp.concatenate([b4, b4r]).reshape(1, 3)
    scal = jnp.stack([alpha1, alpha2, beta1, beta2])

    out = pl.pallas_call(
        _tc_body,
        grid=(NB, NNB),
        in_specs=[
            pl.BlockSpec((1, BN, 8), lambda b, i: (i, b, 0)),
            pl.BlockSpec((BN, 8), lambda b, i: (b, 0)),
            pl.BlockSpec((8, 40), lambda b, i: (0, 0)),
            pl.BlockSpec((1, 40), lambda b, i: (0, 0)),
            pl.BlockSpec((40, 200), lambda b, i: (0, 0)),
            pl.BlockSpec((1, 200), lambda b, i: (0, 0)),
            pl.BlockSpec((200, 40), lambda b, i: (0, 0)),
            pl.BlockSpec((1, 40), lambda b, i: (0, 0)),
            pl.BlockSpec((40, 3), lambda b, i: (0, 0)),
            pl.BlockSpec((1, 3), lambda b, i: (0, 0)),
            pl.BlockSpec(memory_space=pltpu.SMEM),
        ],
        out_specs=pl.BlockSpec((BN, 4), lambda b, i: (b, 0)),
        out_shape=jax.ShapeDtypeStruct((N, 4), jnp.float32),
    )(g, table, w1c, b1c, w2c, b2c, w3c, b3c, w4c, b4c, scal)
    return out


# speedup vs baseline: 3.1588x; 3.0807x over previous
import functools

import jax
import jax.numpy as jnp
import numpy as np
from jax import lax
from jax.experimental import pallas as pl
from jax.experimental.pallas import tpu as pltpu
from jax.experimental.pallas import tpu_sc as plsc

N = 50000
L = 32
NNB = L - 1
E = N * NNB
PI = 3.14159265358
H = float(((2.0 * np.pi) ** 3 / N * L / np.pi / (4.0 / 3.0)) ** (1.0 / 3.0))

NW = 32
EPW = 48440
CH = 9688
NCH = 5

BN = 2048
NB = (N + BN - 1) // BN


def _sc_gather(table, idx):
    mesh = plsc.VectorSubcoreMesh(core_axis_name="c", subcore_axis_name="s")

    @functools.partial(
        pl.kernel,
        out_type=jax.ShapeDtypeStruct((E, 8), jnp.float32),
        mesh=mesh,
        scratch_types=[
            pltpu.VMEM((CH,), jnp.int32),
            pltpu.VMEM((CH, 8), jnp.float32),
            pltpu.SemaphoreType.DMA,
        ],
        compiler_params=pltpu.CompilerParams(use_tc_tiling_on_sc=False),
    )
    def k(table_hbm, idx_hbm, out_hbm, idx_v, rows_v, sem):
        wid = lax.axis_index("s") * 2 + lax.axis_index("c")
        s0 = wid * EPW

        def body(c, carry):
            base = jnp.minimum(s0 + c * CH, E - CH)
            pltpu.sync_copy(idx_hbm.at[pl.ds(base, CH)], idx_v)
            pltpu.async_copy(table_hbm.at[idx_v], rows_v, sem).wait()
            pltpu.sync_copy(rows_v, out_hbm.at[pl.ds(base, CH)])
            return carry

        lax.fori_loop(0, NCH, body, 0)

    return k(table, idx)


def _tc_body(g_ref, t_ref, w1_ref, b1_ref, w2_ref, b2_ref, w3_ref, b3_ref,
             w4_ref, b4_ref, s_ref, o_ref):
    i = pl.program_id(1)
    g = jnp.transpose(g_ref[0])
    t = t_ref[...]

    d = t[0:3] - g[0:3]
    temp1 = jnp.abs(d)
    sgn = -jnp.sign(d) * jnp.sign(d + PI) * jnp.sign(d - PI)
    out = sgn * jnp.minimum(temp1, 2.0 * PI - temp1) / H
    outv = t[3:6] - g[3:6]
    po = out * out
    pv = outv * outv
    pc = out * outv
    out2 = po[0:1] + po[1:2] + po[2:3]
    outv2 = pv[0:1] + pv[1:2] + pv[2:3]
    out2v = pc[0:1] + pc[1:2] + pc[2:3]
    drho1 = t[6:7]
    drho2 = g[6:7]
    sq_out2 = jnp.sqrt(out2)
    sq_outv2 = jnp.sqrt(outv2)

    feat = jnp.concatenate(
        [drho1, drho2, sq_out2, sq_outv2, out2v,
         jnp.zeros((3, BN), jnp.float32)], axis=0)
    r = drho1 - drho2
    dis0 = r / jnp.abs(r)
    disA = out / sq_out2
    disB = outv / sq_outv2

    h1 = jnp.tanh(jnp.dot(w1_ref[...], feat,
                          preferred_element_type=jnp.float32) + b1_ref[...])
    h2 = jnp.tanh(jnp.dot(w2_ref[...], h1,
                          preferred_element_type=jnp.float32) + b2_ref[...])
    h3 = jnp.tanh(jnp.dot(w3_ref[...], h2,
                          preferred_element_type=jnp.float32) + b3_ref[...])
    h4 = jnp.dot(w4_ref[...], h3,
                 preferred_element_type=jnp.float32) + b4_ref[...]

    drho0 = h4[2:3] * dis0
    drhov = h4[0:1] * disA + h4[1:2] * disB

    a1 = jnp.abs(s_ref[0])
    a2 = jnp.abs(s_ref[1])
    bt1 = jnp.abs(s_ref[2])
    bt2 = jnp.abs(s_ref[3])
    denom = out2 + 0.1 * H * H
    out_rho = r * (H * H) / denom
    out_rho = -(bt1 + bt2 * jnp.abs(out_rho)) * out_rho
    o = -1.0 * H * jnp.tanh(-1.0 * out2v) / denom
    o = -a1 * o + a2 * o * o
    drho0 = drho0 + out_rho
    drhov = drhov + o * disA

    delta = jnp.concatenate(
        [drho0, drhov, jnp.zeros((4, BN), jnp.float32)], axis=0)

    @pl.when(i == 0)
    def _():
        o_ref[...] = delta

    @pl.when(i != 0)
    def _():
        o_ref[...] = o_ref[...] + delta


def kernel(X, V, rho, W1, b1, W2, b2, W3, b3, W4, b4, W1r, b1r, W2r, b2r,
           W3r, b3r, W4r, b4r, alpha1, alpha2, beta1, beta2, neighbor, batch):
    del batch

    table = jnp.concatenate([X, V, rho, jnp.zeros((N, 1), jnp.float32)],
                            axis=1)
    idx = jnp.transpose(neighbor[:, 1:]).reshape(-1)

    g_flat = _sc_gather(table, idx)
    g = g_flat.reshape(NNB, N, 8)
    table_t = jnp.transpose(table)

    w1c = jnp.zeros((40, 8), jnp.float32)
    w1c = w1c.at[0:20, 0:5].set(W1.T).at[20:40, 0:5].set(W1r.T)
    b1c = jnp.concatenate([b1, b1r]).reshape(40, 1)
    w2c = jnp.zeros((200, 40), jnp.float32)
    w2c = w2c.at[0:100, 0:20].set(W2.T).at[100:200, 20:40].set(W2r.T)
    b2c = jnp.concatenate([b2, b2r]).reshape(200, 1)
    w3c = jnp.zeros((40, 200), jnp.float32)
    w3c = w3c.at[0:20, 0:100].set(W3.T).at[20:40, 100:200].set(W3r.T)
    b3c = jnp.concatenate([b3, b3r]).reshape(40, 1)
    w4c = jnp.zeros((8, 40), jnp.float32)
    w4c = w4c.at[0:2, 0:20].set(W4.T).at[2:3, 20:40].set(W4r.T)
    b4c = jnp.zeros((8, 1), jnp.float32)
    b4c = b4c.at[0:2, 0].set(b4).at[2, 0].set(b4r[0])
    scal = jnp.stack([alpha1, alpha2, beta1, beta2])

    out_t = pl.pallas_call(
        _tc_body,
        grid=(NB, NNB),
        in_specs=[
            pl.BlockSpec((1, BN, 8), lambda b, i: (i, b, 0)),
            pl.BlockSpec((8, BN), lambda b, i: (0, b)),
            pl.BlockSpec((40, 8), lambda b, i: (0, 0)),
            pl.BlockSpec((40, 1), lambda b, i: (0, 0)),
            pl.BlockSpec((200, 40), lambda b, i: (0, 0)),
            pl.BlockSpec((200, 1), lambda b, i: (0, 0)),
            pl.BlockSpec((40, 200), lambda b, i: (0, 0)),
            pl.BlockSpec((40, 1), lambda b, i: (0, 0)),
            pl.BlockSpec((8, 40), lambda b, i: (0, 0)),
            pl.BlockSpec((8, 1), lambda b, i: (0, 0)),
            pl.BlockSpec(memory_space=pltpu.SMEM),
        ],
        out_specs=pl.BlockSpec((8, BN), lambda b, i: (0, b)),
        out_shape=jax.ShapeDtypeStruct((8, N), jnp.float32),
    )(g, table_t, w1c, b1c, w2c, b2c, w3c, b3c, w4c, b4c, scal)
    return jnp.transpose(out_t[0:4])
